# baseline (speedup 1.0000x reference)
import functools

import jax
import jax.numpy as jnp
from jax import lax
from jax.experimental import pallas as pl
from jax.experimental.pallas import tpu as pltpu
from jax.experimental.pallas import tpu_sc as plsc

_BATCH = 16384
_D = 32
_NW = 32
_BPW = _BATCH // _NW
_CHUNK = 128
_NCHUNK = _BPW // _CHUNK


def _sc_kernel(sid_hbm, pid_hbm, sw_hbm, pw_hbm, out_hbm,
               sidx_v, pidx_v, srows_v, prows_v, out_v, sem):
    wid = lax.axis_index("s") * 2 + lax.axis_index("c")

    pltpu.sync_copy(sid_hbm.at[wid], sidx_v)
    pltpu.sync_copy(pid_hbm.at[wid], pidx_v)

    copies = []
    for j in range(_NCHUNK):
        sl = pl.ds(j * _CHUNK, _CHUNK)
        copies.append(pltpu.make_async_copy(sw_hbm.at[sidx_v.at[j]],
                                            srows_v.at[sl], sem))
        copies.append(pltpu.make_async_copy(pw_hbm.at[pidx_v.at[j]],
                                            prows_v.at[sl], sem))
    for c in copies:
        c.start()
    for c in copies:
        c.wait()

    iota = lax.iota(jnp.int32, 16)

    def body(g, _):
        rowv = g * 16 + iota

        def dstep(acc, d):
            colv = iota + d
            colv = jnp.where(colv >= _D, colv - _D, colv)
            vs = plsc.load_gather(srows_v, [rowv, colv])
            vp = plsc.load_gather(prows_v, [rowv, colv])
            return acc + vs * vp

        acc = jnp.zeros((16,), jnp.float32)
        for d in range(_D):
            acc = dstep(acc, d)
        out_v[pl.ds(g * 16, 16)] = acc
        return 0

    lax.fori_loop(0, _BPW // 16, body, 0)

    pltpu.sync_copy(out_v, out_hbm.at[wid])


def kernel(sid, pid, scientist_weight, paper_weight):
    sid3 = sid.astype(jnp.int32).reshape(_NW, _NCHUNK, _CHUNK)
    pid3 = pid.astype(jnp.int32).reshape(_NW, _NCHUNK, _CHUNK)

    mesh = plsc.VectorSubcoreMesh(core_axis_name="c", subcore_axis_name="s")
    run = pl.kernel(
        _sc_kernel,
        out_type=jax.ShapeDtypeStruct((_NW, _BPW), jnp.float32),
        mesh=mesh,
        scratch_types=[
            pltpu.VMEM((_NCHUNK, _CHUNK), jnp.int32),
            pltpu.VMEM((_NCHUNK, _CHUNK), jnp.int32),
            pltpu.VMEM((_BPW, _D), jnp.float32),
            pltpu.VMEM((_BPW, _D), jnp.float32),
            pltpu.VMEM((_BPW,), jnp.float32),
            pltpu.SemaphoreType.DMA,
        ],
        compiler_params=pltpu.CompilerParams(
            use_tc_tiling_on_sc=False, needs_layout_passes=False),
    )
    out = run(sid3, pid3, scientist_weight, paper_weight)
    return out.reshape(_BATCH)
